# Initial kernel scaffold; baseline (speedup 1.0000x reference)
import jax
import jax.numpy as jnp
from jax.experimental import pallas as pl


def kernel(x_user, x_item, edge_index_ui, edge_index_iu, W1r_ui, W1n_ui, b1_ui, W1r_iu, W1n_iu, b1_iu, W2r_ui, W2n_ui, b2_ui, W2r_iu, W2n_iu, b2_iu, g1_u, be1_u, g1_i, be1_i, g2_u, be2_u, g2_i, be2_i):
    raise NotImplementedError("write your pallas kernel here")


# speedup vs baseline: 3.1439x; 3.1439x over previous
import functools

import jax
import jax.numpy as jnp
from jax import lax
from jax.experimental import pallas as pl
from jax.experimental.pallas import tpu as pltpu
from jax.experimental.pallas import tpu_sc as plsc

N = 10000
D = 128
DH = 256
E = 320000
NC = 2
NS = 16
K = 80
EPS_SUB = E // NS
NCH = EPS_SUB // K
FL_W = 640
FL_S = 624
ZR = 128
ZCH = FL_W // ZR
W = 64
WQ = W


def _fill(ref, rows, cols, value):
    per_row = cols // 16

    def body(i, _):
        r = i // per_row
        c = (i % per_row) * 16
        ref[r, pl.ds(c, 16)] = jnp.full((16,), value, jnp.float32)
        return 0

    lax.fori_loop(0, rows * per_row, body, 0)


def _zero_acc(acc, zb, sid):
    for k in range(ZCH):
        r0 = sid * FL_S + k * ZR
        pltpu.sync_copy(zb, acc.at[pl.ds(r0, ZR)])


def _flush_acc(acc, out_hbm, sid):
    r0 = sid * FL_S
    pltpu.sync_copy(acc.at[pl.ds(r0, FL_W)], out_hbm.at[pl.ds(r0, FL_W)])


def _xform_idx(idxs_v, idxg_v, stride, off):
    per_row = K // 16

    def body(i, _):
        r = i // per_row
        c = (i % per_row) * 16
        idxg_v[r, pl.ds(c, 16)] = idxs_v[r, pl.ds(c, 16)] * stride + off
        return 0

    lax.fori_loop(0, NCH * per_row, body, 0)


def _sc_seg_body(n_tasks, tpc, with_counts, idx_stride, *refs):
    it = iter(refs)
    src_iu, dst_iu, src_ui, dst_ui = (next(it) for _ in range(4))
    tabs = tuple(tuple(next(it) for _ in range(tpc)) for _ in range(2))
    n_out = n_tasks + (1 if with_counts else 0)
    outs = tuple(tuple(next(it) for _ in range(n_out)) for _ in range(2))
    idxs_v = next(it)
    idxd_v = next(it)
    idxg_v = next(it) if idx_stride != 1 else None
    rows_v = next(it)
    ones_v = next(it) if with_counts else None
    zb = next(it)
    acc = next(it)
    sem = next(it)

    cid = lax.axis_index("c")
    sid = lax.axis_index("s")

    _fill(zb, ZR, W, 0.0)
    if with_counts:
        _fill(ones_v, K, W, 1.0)

    for t in range(n_tasks + (1 if with_counts else 0)):
        counts_now = t == n_tasks
        _zero_acc(acc, zb, sid)
        plsc.subcore_barrier()

        for c, (src_3d, dst_3d) in enumerate(((src_iu, dst_iu),
                                              (src_ui, dst_ui))):
            @pl.when(cid == c)
            def _(c=c, src_3d=src_3d, dst_3d=dst_3d, t=t,
                  counts_now=counts_now):
                if t == 0:
                    pltpu.sync_copy(src_3d.at[sid], idxs_v)
                    pltpu.sync_copy(dst_3d.at[sid], idxd_v)
                if counts_now:
                    def body(j, _):
                        pltpu.sync_copy(ones_v, acc.at[idxd_v.at[j]],
                                        add=True)
                        return 0
                else:
                    if idx_stride != 1:
                        _xform_idx(idxs_v, idxg_v, idx_stride, t)
                    gidx = idxg_v if idx_stride != 1 else idxs_v
                    tab = tabs[c][t % tpc]

                    def body(j, _):
                        pltpu.async_copy(tab.at[gidx.at[j]], rows_v,
                                         sem).wait()
                        pltpu.sync_copy(rows_v, acc.at[idxd_v.at[j]],
                                        add=True)
                        return 0

                lax.fori_loop(0, NCH, body, 0)

        plsc.subcore_barrier()

        for c in range(2):
            @pl.when(cid == c)
            def _(c=c, t=t):
                _flush_acc(acc, outs[c][t], sid)

        plsc.subcore_barrier()


RB = 2000
NG = N // RB


def _tc_matmul_stats(n_s, n_x, *refs):
    s_refs = refs[:n_s]
    x_refs = refs[n_s:n_s + n_x]
    cnt_ref, wn_ref, wr_ref, b_ref, z_ref, stats_ref, acc_ref = \
        refs[n_s + n_x:]
    rcp = 1.0 / jnp.maximum(cnt_ref[:, 0:1], 1.0)
    z = jnp.broadcast_to(b_ref[...], (RB, DH))
    for q, s_ref in enumerate(s_refs):
        z = z + jnp.dot(s_ref[...] * rcp, wn_ref[q * W:(q + 1) * W, :],
                        preferred_element_type=jnp.float32)
    off = 0
    for x_ref in x_refs:
        xw = x_ref.shape[1]
        z = z + jnp.dot(x_ref[...], wr_ref[off:off + xw, :],
                        preferred_element_type=jnp.float32)
        off += xw
    z_ref[...] = z
    i = pl.program_id(0)

    @pl.when(i == 0)
    def _():
        acc_ref[...] = jnp.zeros_like(acc_ref)

    acc_ref[0:1, :] += jnp.sum(z, axis=0, keepdims=True)
    acc_ref[1:2, :] += jnp.sum(z * z, axis=0, keepdims=True)

    @pl.when(i == NG - 1)
    def _():
        stats_ref[...] = acc_ref[...]


def _tc_bn_elu(n_out, ow, z_ref, stats_ref, g_ref, be_ref, *o_refs):
    m = stats_ref[0:1, :] * (1.0 / N)
    v = stats_ref[1:2, :] * (1.0 / N) - m * m
    z = z_ref[...]
    zn = g_ref[...] * (z - m) / jnp.sqrt(v + 1e-5) + be_ref[...]
    h = jnp.where(zn > 0, zn, jnp.exp(zn) - 1.0)
    for q, o_ref in enumerate(o_refs):
        o_ref[...] = h[:, q * ow:(q + 1) * ow]


def _tc_dense(s_list, cnt, x_list, wn, wr, b, g, be, n_out, ow):
    f32 = jnp.float32
    n_s = len(s_list)
    blk = lambda w: pl.BlockSpec((RB, w), lambda i: (i, 0))
    full = lambda a: pl.BlockSpec(a.shape, lambda i: tuple(0 for _ in a.shape))
    z, stats = pl.pallas_call(
        functools.partial(_tc_matmul_stats, n_s, len(x_list)),
        grid=(NG,),
        in_specs=([blk(W)] * n_s
                  + [blk(x.shape[1]) for x in x_list]
                  + [blk(W), full(wn), full(wr), full(b)]),
        out_specs=(blk(DH), pl.BlockSpec((2, DH), lambda i: (0, 0))),
        out_shape=(jax.ShapeDtypeStruct((N, DH), f32),
                   jax.ShapeDtypeStruct((2, DH), f32)),
        scratch_shapes=[pltpu.VMEM((2, DH), f32)],
    )(*s_list, *x_list, cnt, wn, wr, b)
    outs = pl.pallas_call(
        functools.partial(_tc_bn_elu, n_out, ow),
        grid=(NG,),
        in_specs=[blk(DH), pl.BlockSpec((2, DH), lambda i: (0, 0)),
                  full(g), full(be)],
        out_specs=tuple(blk(ow) for _ in range(n_out)),
        out_shape=tuple(
            jax.ShapeDtypeStruct((N, ow), f32) for _ in range(n_out)),
    )(z, stats, g, be)
    return outs


def _make_sc_seg(n_tasks, tpc, with_counts, idx_stride):
    f32 = jnp.float32
    n_out = n_tasks + (1 if with_counts else 0)
    out_type = []
    for _ in range(2):
        out_type += [jax.ShapeDtypeStruct((N, W), f32)] * n_out
    scratch = [
        pltpu.VMEM((NCH, K), jnp.int32),
        pltpu.VMEM((NCH, K), jnp.int32),
    ]
    if idx_stride != 1:
        scratch.append(pltpu.VMEM((NCH, K), jnp.int32))
    scratch.append(pltpu.VMEM((K, W), f32))
    if with_counts:
        scratch.append(pltpu.VMEM((K, W), f32))
    scratch.append(pltpu.VMEM((ZR, W), f32))
    scratch.append(pltpu.VMEM_SHARED((N, W), f32))
    scratch.append(pltpu.SemaphoreType.DMA)
    return pl.kernel(
        functools.partial(_sc_seg_body, n_tasks, tpc, with_counts,
                          idx_stride),
        out_type=tuple(out_type),
        mesh=plsc.VectorSubcoreMesh(core_axis_name="c", subcore_axis_name="s"),
        compiler_params=pltpu.CompilerParams(use_tc_tiling_on_sc=False),
        scratch_types=scratch,
    )


def kernel(x_user, x_item, edge_index_ui, edge_index_iu,
           W1r_ui, W1n_ui, b1_ui, W1r_iu, W1n_iu, b1_iu,
           W2r_ui, W2n_ui, b2_ui, W2r_iu, W2n_iu, b2_iu,
           g1_u, be1_u, g1_i, be1_i, g2_u, be2_u, g2_i, be2_i):
    f32 = jnp.float32
    ei_ui = edge_index_ui.astype(jnp.int32)
    ei_iu = edge_index_iu.astype(jnp.int32)
    src_ui = ei_ui[0].reshape(NS, NCH, K)
    dst_ui = ei_ui[1].reshape(NS, NCH, K)
    src_iu = ei_iu[0].reshape(NS, NCH, K)
    dst_iu = ei_iu[1].reshape(NS, NCH, K)

    x_item2 = x_item.reshape(2 * N, W)
    x_user2 = x_user.reshape(2 * N, W)
    s1u_h0, s1u_h1, cnt_u, s1i_h0, s1i_h1, cnt_i = _make_sc_seg(
        2, 1, True, 2)(src_iu, dst_iu, src_ui, dst_ui, x_item2, x_user2)

    row = lambda a: a.reshape(1, -1)
    h1u_q = _tc_dense([s1u_h0, s1u_h1], cnt_u, [x_user], W1n_iu, W1r_iu,
                      row(b1_iu), row(g1_u), row(be1_u), 4, WQ)
    h1i_q = _tc_dense([s1i_h0, s1i_h1], cnt_i, [x_item], W1n_ui, W1r_ui,
                      row(b1_ui), row(g1_i), row(be1_i), 4, WQ)

    s2 = _make_sc_seg(4, 4, False, 1)(
        src_iu, dst_iu, src_ui, dst_ui, *h1i_q, *h1u_q)
    s2u_q, s2i_q = s2[:4], s2[4:]

    h2u_h0, h2u_h1 = _tc_dense(list(s2u_q), cnt_u, list(h1u_q), W2n_iu,
                               W2r_iu, row(b2_iu), row(g2_u), row(be2_u),
                               2, D)
    h2i_h0, h2i_h1 = _tc_dense(list(s2i_q), cnt_i, list(h1i_q), W2n_ui,
                               W2r_ui, row(b2_ui), row(g2_i), row(be2_i),
                               2, D)

    h1_u = jnp.concatenate(h1u_q, axis=1)
    h1_i = jnp.concatenate(h1i_q, axis=1)
    h2_u = jnp.concatenate([h2u_h0, h2u_h1], axis=1)
    h2_i = jnp.concatenate([h2i_h0, h2i_h1], axis=1)
    return (x_user, x_item, h1_u, h1_i, h2_u, h2_i)


# speedup vs baseline: 4.4677x; 1.4211x over previous
import functools

import jax
import jax.numpy as jnp
from jax import lax
from jax.experimental import pallas as pl
from jax.experimental.pallas import tpu as pltpu
from jax.experimental.pallas import tpu_sc as plsc

N = 10000
D = 128
DH = 256
E = 320000
NC = 2
NS = 16
K = 48
EPS_SUB = 20160
NCH = EPS_SUB // K
NACC = N + 8
FL_W = 640
FL_S = 624
ZR = 128
ZCH = FL_W // ZR
W = 64
NBUF = 5
GLEAD = 3
SLAG = 2
RB = 2000
NG = N // RB


def _fill(ref, rows, cols, value):
    per_row = cols // 16

    def body(i, _):
        r = i // per_row
        c = (i % per_row) * 16
        ref[r, pl.ds(c, 16)] = jnp.full((16,), value, jnp.float32)
        return 0

    lax.fori_loop(0, rows * per_row, body, 0)


def _xform_idx(idxs_v, idxg_v, stride, off):
    per_row = K // 16

    def body(i, _):
        r = i // per_row
        c = (i % per_row) * 16
        idxg_v[r, pl.ds(c, 16)] = idxs_v[r, pl.ds(c, 16)] * stride + off
        return 0

    lax.fori_loop(0, NCH * per_row, body, 0)


def _zero_acc(acc, zb, sid):
    for k in range(ZCH):
        pltpu.sync_copy(zb, acc.at[pl.ds(sid * FL_S + k * ZR, ZR)])


def _sc_seg_body(n_tasks, with_counts, stride, core_span, task_span, *refs):
    (src_all, dst_all, table, out, idxs_v, idxd_v, idxg_v) = refs[:7]
    rows_b = refs[7:7 + NBUF]
    rest = refs[7 + NBUF:]
    if with_counts:
        ones_v, zb, acc = rest[:3]
        rest = rest[3:]
    else:
        zb, acc = rest[:2]
        rest = rest[2:]
        ones_v = None
    sem_g = rest[:NBUF]
    sem_s = rest[NBUF:2 * NBUF]

    cid = lax.axis_index("c")
    sid = lax.axis_index("s")

    _fill(zb, ZR, W, 0.0)
    if with_counts:
        _fill(ones_v, K, W, 1.0)
    pltpu.sync_copy(src_all.at[cid, sid], idxs_v)
    pltpu.sync_copy(dst_all.at[cid, sid], idxd_v)

    def run_task(t, gather):
        _zero_acc(acc, zb, sid)
        plsc.subcore_barrier()

        if gather:
            _xform_idx(idxs_v, idxg_v, stride,
                       cid * core_span + t * task_span)
            for b in range(GLEAD):
                pltpu.async_copy(table.at[idxg_v.at[b]], rows_b[b],
                                 sem_g[b])

            def body(g, _):
                for b in range(NBUF):
                    j = g * NBUF + b
                    pltpu.make_async_copy(table.at[idxg_v.at[j]],
                                          rows_b[b], sem_g[b]).wait()
                    pltpu.async_copy(rows_b[b], acc.at[idxd_v.at[j]],
                                     sem_s[b], add=True)

                    @pl.when(j >= SLAG)
                    def _(j=j, b=b):
                        b2 = (b - SLAG) % NBUF
                        pltpu.make_async_copy(
                            rows_b[b2], acc.at[idxd_v.at[j - SLAG]],
                            sem_s[b2]).wait()

                    @pl.when(j + GLEAD < NCH)
                    def _(j=j, b=b):
                        b3 = (b + GLEAD) % NBUF
                        pltpu.async_copy(table.at[idxg_v.at[j + GLEAD]],
                                         rows_b[b3], sem_g[b3])
                return 0

            lax.fori_loop(0, NCH // NBUF, body, 0)
            for jj in range(NCH - SLAG, NCH):
                pltpu.make_async_copy(rows_b[jj % NBUF],
                                      acc.at[idxd_v.at[jj]],
                                      sem_s[jj % NBUF]).wait()
        else:
            def body(j, _):
                pltpu.async_copy(ones_v, acc.at[idxd_v.at[j]],
                                 sem_s[0], add=True)

                @pl.when(j >= NBUF - 1)
                def _():
                    pltpu.make_async_copy(
                        ones_v, acc.at[idxd_v.at[j]], sem_s[0]).wait()
                return 0

            lax.fori_loop(0, NCH, body, 0)

            def drain(j, _):
                pltpu.make_async_copy(ones_v, acc.at[idxd_v.at[j]],
                                      sem_s[0]).wait()
                return 0

            lax.fori_loop(0, NBUF - 1, drain, 0)

        plsc.subcore_barrier()
        r0 = sid * FL_S
        pltpu.sync_copy(acc.at[pl.ds(r0, FL_W)],
                        out.at[cid, t, pl.ds(r0, FL_W)])
        plsc.subcore_barrier()

    def task_body(t, _):
        run_task(t, True)
        return 0

    lax.fori_loop(0, n_tasks, task_body, 0)
    if with_counts:
        run_task(n_tasks, False)


def _make_sc_seg(n_tasks, with_counts, stride, core_span, task_span,
                 table_rows):
    f32 = jnp.float32
    n_out = n_tasks + (1 if with_counts else 0)
    scratch = [
        pltpu.VMEM((NCH, K), jnp.int32),
        pltpu.VMEM((NCH, K), jnp.int32),
        pltpu.VMEM((NCH, K), jnp.int32),
    ]
    scratch += [pltpu.VMEM((K, W), f32)] * NBUF
    if with_counts:
        scratch.append(pltpu.VMEM((K, W), f32))
    scratch.append(pltpu.VMEM((ZR, W), f32))
    scratch.append(pltpu.VMEM_SHARED((NACC, W), f32))
    scratch += [pltpu.SemaphoreType.DMA] * (2 * NBUF)
    return pl.kernel(
        functools.partial(_sc_seg_body, n_tasks, with_counts, stride,
                          core_span, task_span),
        out_type=jax.ShapeDtypeStruct((NC, n_out, N, W), f32),
        mesh=plsc.VectorSubcoreMesh(core_axis_name="c", subcore_axis_name="s"),
        compiler_params=pltpu.CompilerParams(use_tc_tiling_on_sc=False),
        scratch_types=scratch,
    )


def _tc_mm_body(n_s, s_ref, cnt_ref, x_ref, wn_ref, wr_ref, b_ref,
                z_ref, stats_ref, acc_ref):
    rcp = 1.0 / jnp.maximum(cnt_ref[0, 0, :, 0:1], 1.0)
    z = jnp.broadcast_to(b_ref[0], (RB, DH))
    for q in range(n_s):
        z = z + jnp.dot(s_ref[0, q] * rcp, wn_ref[0, q * W:(q + 1) * W, :],
                        preferred_element_type=jnp.float32)
    z = z + jnp.dot(x_ref[0], wr_ref[0], preferred_element_type=jnp.float32)
    z_ref[0] = z
    i = pl.program_id(1)

    @pl.when(i == 0)
    def _():
        acc_ref[...] = jnp.zeros_like(acc_ref)

    acc_ref[0:1, :] += jnp.sum(z, axis=0, keepdims=True)
    acc_ref[1:2, :] += jnp.sum(z * z, axis=0, keepdims=True)

    @pl.when(i == NG - 1)
    def _():
        stats_ref[0] = acc_ref[...]


def _tc_bn_body(z_ref, stats_ref, g_ref, be_ref, o_ref):
    m = stats_ref[0, 0:1, :] * (1.0 / N)
    v = stats_ref[0, 1:2, :] * (1.0 / N) - m * m
    z = z_ref[0]
    zn = g_ref[0] * (z - m) / jnp.sqrt(v + 1e-5) + be_ref[0]
    o_ref[0] = jnp.where(zn > 0, zn, jnp.exp(zn) - 1.0)


def _tc_dense(s_all, n_s, cnt_all, cnt_slot, x, x_flip, wn, wr, b, g, be,
              out_flip):
    f32 = jnp.float32
    xw = x.shape[-1]
    xsel = (lambda t: 1 - t) if x_flip else (lambda t: t)
    osel = (lambda t: 1 - t) if out_flip else (lambda t: t)
    z, stats = pl.pallas_call(
        functools.partial(_tc_mm_body, n_s),
        grid=(2, NG),
        in_specs=[
            pl.BlockSpec((1, n_s, RB, W), lambda t, i: (t, 0, i, 0)),
            pl.BlockSpec((1, 1, RB, W), lambda t, i: (t, cnt_slot, i, 0)),
            pl.BlockSpec((1, RB, xw), lambda t, i: (xsel(t), i, 0)),
            pl.BlockSpec((1, n_s * W, DH), lambda t, i: (t, 0, 0)),
            pl.BlockSpec((1, xw, DH), lambda t, i: (t, 0, 0)),
            pl.BlockSpec((1, 1, DH), lambda t, i: (t, 0, 0)),
        ],
        out_specs=(pl.BlockSpec((1, RB, DH), lambda t, i: (t, i, 0)),
                   pl.BlockSpec((1, 2, DH), lambda t, i: (t, 0, 0))),
        out_shape=(jax.ShapeDtypeStruct((2, N, DH), f32),
                   jax.ShapeDtypeStruct((2, 2, DH), f32)),
        scratch_shapes=[pltpu.VMEM((2, DH), f32)],
    )(s_all, cnt_all, x, wn, wr, b)
    return pl.pallas_call(
        _tc_bn_body,
        grid=(2, NG),
        in_specs=[
            pl.BlockSpec((1, RB, DH), lambda t, i: (t, i, 0)),
            pl.BlockSpec((1, 2, DH), lambda t, i: (t, 0, 0)),
            pl.BlockSpec((1, 1, DH), lambda t, i: (t, 0, 0)),
            pl.BlockSpec((1, 1, DH), lambda t, i: (t, 0, 0)),
        ],
        out_specs=pl.BlockSpec((1, RB, DH), lambda t, i: (osel(t), i, 0)),
        out_shape=jax.ShapeDtypeStruct((2, N, DH), f32),
    )(z, stats, g, be)


def kernel(x_user, x_item, edge_index_ui, edge_index_iu,
           W1r_ui, W1n_ui, b1_ui, W1r_iu, W1n_iu, b1_iu,
           W2r_ui, W2n_ui, b2_ui, W2r_iu, W2n_iu, b2_iu,
           g1_u, be1_u, g1_i, be1_i, g2_u, be2_u, g2_i, be2_i):
    ei_ui = edge_index_ui.astype(jnp.int32)
    ei_iu = edge_index_iu.astype(jnp.int32)

    pad_n = EPS_SUB - E // NS

    def shard(row, fill):
        p = jnp.full((NS, pad_n), fill, jnp.int32)
        return jnp.concatenate([row.reshape(NS, E // NS), p], axis=1)

    src_all = jnp.stack([shard(ei_iu[0], 0), shard(ei_ui[0], 0)])
    src_all = src_all.reshape(NC, NS, NCH, K)
    dst_all = jnp.stack([shard(ei_iu[1], N), shard(ei_ui[1], N)])
    dst_all = dst_all.reshape(NC, NS, NCH, K)

    stk = lambda a, bb: jnp.stack([a, bb])
    row2 = lambda a, bb: jnp.stack([a, bb]).reshape(2, 1, DH)

    x_flat1 = jnp.concatenate([x_item, x_user], axis=0).reshape(4 * N, W)
    sc1 = _make_sc_seg(2, True, 2, 2 * N, 1, 4 * N)(src_all, dst_all,
                                                    x_flat1)

    h1 = _tc_dense(sc1, 2, sc1, 2, stk(x_user, x_item), False,
                   stk(W1n_iu, W1n_ui), stk(W1r_iu, W1r_ui),
                   row2(b1_iu, b1_ui), row2(g1_u, g1_i),
                   row2(be1_u, be1_i), True)

    sc2 = _make_sc_seg(4, False, 4, 4 * N, 1, 8 * N)(
        src_all, dst_all, h1.reshape(8 * N, W))

    h2 = _tc_dense(sc2, 4, sc1, 2, h1, True,
                   stk(W2n_iu, W2n_ui), stk(W2r_iu, W2r_ui),
                   row2(b2_iu, b2_ui), row2(g2_u, g2_i),
                   row2(be2_u, be2_i), False)

    return (x_user, x_item, h1[1], h1[0], h2[0], h2[1])
